# baseline (speedup 1.0000x reference)
import functools

import numpy as _np

import jax
import jax.numpy as jnp
from jax import lax
from jax.experimental import pallas as pl
from jax.experimental.pallas import tpu as pltpu
from jax.experimental.pallas import tpu_sc as plsc

N_NODES = 10000
NPAD = 10240
E_EDGES = 320000
HID = 128
HEADS = 8
DH = 16
WACC = 144
NCLS = 10

NC = 2
NS = 16
EPT = 10112
EPAD = EPT * NC * NS
CHK = 64
IDXB = 2
NCHUNK = EPT // CHK
ROWS_PT = NPAD // NS

_f32 = jnp.float32
_i32 = jnp.int32

_sc_mesh = plsc.VectorSubcoreMesh(core_axis_name="c", subcore_axis_name="s")


def _shuf(v, idx):
    dnums = lax.GatherDimensionNumbers(
        offset_dims=(), collapsed_slice_dims=(0,), start_index_map=(0,))
    return lax.gather(v, idx[:, None], dnums, (1,),
                      mode=lax.GatherScatterMode.PROMISE_IN_BOUNDS)


@functools.partial(
    pl.kernel,
    out_type=jax.ShapeDtypeStruct((NPAD, HID), _f32),
    mesh=_sc_mesh,
    scratch_types=[
        pltpu.VMEM((80,), _i32),
        pltpu.VMEM((80, HID), _f32),
        pltpu.SemaphoreType.DMA,
    ],
)
def _emb_gather(nf_hbm, emb_hbm, out_hbm, idx_v, rows_v, sem):
    wid = lax.axis_index("c") * NS + lax.axis_index("s")
    for j in range(4):
        base = wid * 320 + j * 80
        pltpu.sync_copy(nf_hbm.at[pl.ds(base, 80)], idx_v)
        pltpu.async_copy(emb_hbm.at[idx_v], rows_v, sem).wait()
        pltpu.sync_copy(rows_v, out_hbm.at[pl.ds(base, 80)])


@functools.partial(
    pl.kernel,
    out_type=jax.ShapeDtypeStruct((NC, NPAD, WACC), _f32),
    mesh=_sc_mesh,
    scratch_types=[
        pltpu.VMEM((2, IDXB, CHK), _i32),
        pltpu.VMEM((2, IDXB, CHK), _i32),
        pltpu.VMEM((2, CHK, HID), jnp.bfloat16),
        pltpu.VMEM((2, CHK, HID), jnp.bfloat16),
        pltpu.VMEM((2, CHK, HID), jnp.bfloat16),
        pltpu.VMEM((CHK, WACC), _f32),
        pltpu.VMEM_SHARED((NPAD, WACC), _f32),
        pltpu.SemaphoreType.DMA,
        pltpu.SemaphoreType.DMA,
    ],
    compiler_params=pltpu.CompilerParams(use_tc_tiling_on_sc=False, needs_layout_passes=False),
)
def _edge_attn(q_hbm, k_hbm, v_hbm, src_hbm, dst_hbm, out_hbm,
               sidx, didx, kbuf, qbuf, vbuf, wbuf, acc, gsem0, gsem1):
    cid = lax.axis_index("c")
    sid = lax.axis_index("s")
    zero16 = jnp.zeros((16,), _f32)
    iota16 = lax.iota(_i32, 16)
    perms = [jnp.bitwise_xor(iota16, k2) for k2 in (4, 2, 1)]
    ppick = (iota16 & 1) * 8
    pmask = [lax.shift_right_logical(iota16, 1) == g for g in range(4)]
    gsem = (gsem0, gsem1)

    row0 = sid * ROWS_PT
    rbase = (cid * NS + sid) * NCHUNK

    def _stage(sl, row, b):
        pltpu.async_copy(k_hbm.at[sidx.at[sl, row]], kbuf.at[b], gsem[b])
        pltpu.async_copy(q_hbm.at[didx.at[sl, row]], qbuf.at[b], gsem[b])
        pltpu.async_copy(v_hbm.at[sidx.at[sl, row]], vbuf.at[b], gsem[b])

    def _drain(b):
        pltpu.make_async_copy(k_hbm.at[pl.ds(0, CHK)], kbuf.at[b], gsem[b]).wait()
        pltpu.make_async_copy(q_hbm.at[pl.ds(0, CHK)], qbuf.at[b], gsem[b]).wait()
        pltpu.make_async_copy(v_hbm.at[pl.ds(0, CHK)], vbuf.at[b], gsem[b]).wait()

    def _load_idx(jb, sl):
        pltpu.sync_copy(src_hbm.at[pl.ds(rbase + jb * IDXB, IDXB)], sidx.at[sl])
        pltpu.sync_copy(dst_hbm.at[pl.ds(rbase + jb * IDXB, IDXB)], didx.at[sl])
    _load_idx(0, 0)
    _stage(0, 0, 0)

    def _zrow(r, carry):
        for j in range(WACC // 16):
            wbuf[r, pl.ds(j * 16, 16)] = zero16
        return carry
    lax.fori_loop(0, CHK, _zrow, 0)
    def _zacc(j, carry):
        pltpu.sync_copy(wbuf, acc.at[pl.ds(row0 + j * CHK, CHK)])
        return carry
    lax.fori_loop(0, ROWS_PT // CHK, _zacc, 0)
    plsc.subcore_barrier()


    def _batch(jb, carry):
        sl = jb % 2
        nsl = 1 - sl
        _load_idx(jb + 1, nsl)
        for j in range(IDXB):
            b = j & 1
            if j < IDXB - 1:
                _stage(sl, j + 1, 1 - b)
            else:
                _stage(nsl, 0, 1 - b)
            _drain(b)

            @plsc.parallel_loop(0, CHK, 1, unroll=2)
            def _edge(e):
                drow = jnp.zeros((16,), _f32)
                for g in range(HEADS // 2):
                    c0 = g * 32
                    klo, khi = plsc.unpack(kbuf[b, e, pl.ds(c0, 32)],
                                           format=plsc.PackFormat.INTERLEAVED,
                                           preferred_element_type=_f32)
                    qlo, qhi = plsc.unpack(qbuf[b, e, pl.ds(c0, 32)],
                                           format=plsc.PackFormat.INTERLEAVED,
                                           preferred_element_type=_f32)
                    sv = klo * qlo + khi * qhi
                    for p2 in perms:
                        sv = sv + _shuf(sv, p2)
                    sv = jnp.exp(jnp.clip(sv, -5.0, 5.0))
                    vlo, vhi = plsc.unpack(vbuf[b, e, pl.ds(c0, 32)],
                                           format=plsc.PackFormat.INTERLEAVED,
                                           preferred_element_type=_f32)
                    wbuf[e, pl.ds(c0, 16)] = sv * vlo
                    wbuf[e, pl.ds(c0 + 16, 16)] = sv * vhi
                    drow = jnp.where(pmask[g], _shuf(sv, ppick), drow)
                wbuf[e, pl.ds(HID, 16)] = drow
            pltpu.sync_copy(wbuf, acc.at[didx.at[sl, j]], add=True)
        return carry
    lax.fori_loop(0, NCHUNK // IDXB, _batch, 0)
    _drain(0)

    plsc.subcore_barrier()
    def _wout(j, carry):
        r = row0 + j * CHK
        pltpu.sync_copy(acc.at[pl.ds(r, CHK)], out_hbm.at[cid, pl.ds(r, CHK)])
        return carry
    lax.fori_loop(0, ROWS_PT // CHK, _wout, 0)


BLK = 1024


def _qkv_body(h_ref, wq, wk, wv, bq, bk, bv, q_ref, k_ref, v_ref):
    h = h_ref[...]
    bf = jnp.bfloat16
    q_ref[...] = (jnp.dot(h, wq[...], preferred_element_type=_f32) + bq[...]).astype(bf)
    k_ref[...] = (jnp.dot(h, wk[...], preferred_element_type=_f32) + bk[...]).astype(bf)
    v_ref[...] = (jnp.dot(h, wv[...], preferred_element_type=_f32) + bv[...]).astype(bf)


def _qkv_call(h, wq, wk, wv, bq, bk, bv):
    grid = (NPAD // BLK,)
    row_spec = pl.BlockSpec((BLK, HID), lambda i: (i, 0))
    w_spec = pl.BlockSpec((HID, HID), lambda i: (0, 0))
    b_spec = pl.BlockSpec((1, HID), lambda i: (0, 0))
    return pl.pallas_call(
        _qkv_body,
        grid=grid,
        in_specs=[row_spec, w_spec, w_spec, w_spec, b_spec, b_spec, b_spec],
        out_specs=[row_spec, row_spec, row_spec],
        out_shape=[jax.ShapeDtypeStruct((NPAD, HID), jnp.bfloat16)] * 3,
    )(h, wq, wk, wv, bq, bk, bv)


def _ln(x, g, b):
    mu = jnp.mean(x, axis=1, keepdims=True)
    xc = x - mu
    var = jnp.mean(xc * xc, axis=1, keepdims=True)
    return xc * lax.rsqrt(var + 1e-5) * g + b


def _post_h(n0, d0, n1, d1, hin, rexp, wo, bo, w1, b1, w2, b2, g1, be1, g2, be2):
    numer = n0[...] + n1[...]
    den = d0[...] + d1[...] + 1e-6
    denexp = jnp.dot(den, rexp[...], preferred_element_type=_f32)
    attn = numer / denexp
    h = hin[...] + jnp.dot(attn, wo[...], preferred_element_type=_f32) + bo[...]
    h = _ln(h, g1[...], be1[...])
    ff = jnp.maximum(jnp.dot(h, w1[...], preferred_element_type=_f32) + b1[...], 0.0)
    h2 = h + jnp.dot(ff, w2[...], preferred_element_type=_f32) + b2[...]
    return _ln(h2, g2[...], be2[...])


def _post_qkv_body(n0, d0, n1, d1, hin, rexp, wo, bo, w1, b1, w2, b2, g1, be1,
                   g2, be2, wq, wk, wv, bq, bk, bv, h_ref, q_ref, k_ref, v_ref):
    h = _post_h(n0, d0, n1, d1, hin, rexp, wo, bo, w1, b1, w2, b2, g1, be1, g2, be2)
    h_ref[...] = h
    bf = jnp.bfloat16
    q_ref[...] = (jnp.dot(h, wq[...], preferred_element_type=_f32) + bq[...]).astype(bf)
    k_ref[...] = (jnp.dot(h, wk[...], preferred_element_type=_f32) + bk[...]).astype(bf)
    v_ref[...] = (jnp.dot(h, wv[...], preferred_element_type=_f32) + bv[...]).astype(bf)


def _post_readout_body(n0, d0, n1, d1, hin, rexp, wo, bo, w1, b1, w2, b2, g1, be1,
                       g2, be2, m0, c0, m1, c1, m2, c2, out_ref):
    h = _post_h(n0, d0, n1, d1, hin, rexp, wo, bo, w1, b1, w2, b2, g1, be1, g2, be2)
    x = jnp.maximum(jnp.dot(h, m0[...], preferred_element_type=_f32) + c0[...], 0.0)
    x = jnp.maximum(jnp.dot(x, m1[...], preferred_element_type=_f32) + c1[...], 0.0)
    out_ref[...] = jnp.dot(x, m2[...], preferred_element_type=_f32) + c2[...]


def _post_args(lp):
    return (lp['Wo'], lp['bo'].reshape(1, HID), lp['W1'], lp['b1'].reshape(1, 2 * HID),
            lp['W2'], lp['b2'].reshape(1, HID), lp['g1'].reshape(1, HID),
            lp['be1'].reshape(1, HID), lp['g2'].reshape(1, HID), lp['be2'].reshape(1, HID))


def _post_specs():
    row_spec = pl.BlockSpec((BLK, HID), lambda i: (i, 0))
    den_spec = pl.BlockSpec((BLK, 16), lambda i: (i, 0))
    rexp_spec = pl.BlockSpec((16, HID), lambda i: (0, 0))
    w_spec = pl.BlockSpec((HID, HID), lambda i: (0, 0))
    w1_spec = pl.BlockSpec((HID, 2 * HID), lambda i: (0, 0))
    w2_spec = pl.BlockSpec((2 * HID, HID), lambda i: (0, 0))
    b_spec = pl.BlockSpec((1, HID), lambda i: (0, 0))
    b1_spec = pl.BlockSpec((1, 2 * HID), lambda i: (0, 0))
    return [row_spec, den_spec, row_spec, den_spec, row_spec, rexp_spec,
            w_spec, b_spec, w1_spec, b1_spec, w2_spec, b_spec,
            b_spec, b_spec, b_spec, b_spec]


def _post_qkv_call(n0, d0, n1, d1, hin, rexp, lp, nlp):
    grid = (NPAD // BLK,)
    row_spec = pl.BlockSpec((BLK, HID), lambda i: (i, 0))
    w_spec = pl.BlockSpec((HID, HID), lambda i: (0, 0))
    b_spec = pl.BlockSpec((1, HID), lambda i: (0, 0))
    scale = 0.25
    return pl.pallas_call(
        _post_qkv_body,
        grid=grid,
        in_specs=_post_specs() + [w_spec, w_spec, w_spec, b_spec, b_spec, b_spec],
        out_specs=[row_spec, row_spec, row_spec, row_spec],
        out_shape=[jax.ShapeDtypeStruct((NPAD, HID), _f32)] +
                  [jax.ShapeDtypeStruct((NPAD, HID), jnp.bfloat16)] * 3,
    )(n0, d0, n1, d1, hin, rexp, *_post_args(lp),
      nlp['Wq'] * scale, nlp['Wk'], nlp['Wv'],
      (nlp['bq'] * scale).reshape(1, HID), nlp['bk'].reshape(1, HID),
      nlp['bv'].reshape(1, HID))


def _post_readout_call(n0, d0, n1, d1, hin, rexp, lp, mlp):
    grid = (NPAD // BLK,)
    row_spec = pl.BlockSpec((BLK, HID), lambda i: (i, 0))
    w_spec = pl.BlockSpec((HID, HID), lambda i: (0, 0))
    b_spec = pl.BlockSpec((1, HID), lambda i: (0, 0))
    (w0, b0), (w1, b1), (w2, b2) = mlp
    return pl.pallas_call(
        _post_readout_body,
        grid=grid,
        in_specs=_post_specs() + [w_spec, b_spec, w_spec, b_spec, w_spec, b_spec],
        out_specs=row_spec,
        out_shape=jax.ShapeDtypeStruct((NPAD, HID), _f32),
    )(n0, d0, n1, d1, hin, rexp, *_post_args(lp),
      _pad_mat(w0, HID, HID), _pad_mat(b0.reshape(1, -1), 1, HID),
      _pad_mat(w1, HID, HID), _pad_mat(b1.reshape(1, -1), 1, HID),
      _pad_mat(w2, HID, HID), _pad_mat(b2.reshape(1, -1), 1, HID))


def _pad_mat(w, rows, cols):
    r, c = w.shape
    return jnp.pad(w, ((0, rows - r), (0, cols - c)))


def kernel(node_feat, edge_index, params):
    src = edge_index[0]
    dst = edge_index[1]
    pad_n = EPAD + IDXB * CHK - E_EDGES
    src_p = jnp.concatenate([src, jnp.full((pad_n,), NPAD - 1, _i32)]).reshape(-1, CHK)
    dst_p = jnp.concatenate([dst, jnp.full((pad_n,), NPAD - 1, _i32)]).reshape(-1, CHK)
    nf_p = jnp.concatenate([node_feat, jnp.zeros((NPAD - N_NODES,), _i32)])

    c = _np.arange(HID)
    g2 = c // 32
    r = c % 32
    head = 2 * g2 + ((r // 8) & 1)
    feat = 2 * (r % 8) + (r // 16)
    o_idx = head * DH + feat
    rexp = jnp.asarray((_np.arange(16)[:, None] == head[None, :]).astype(_np.float32))

    h = _emb_gather(nf_p, params['emb'])

    layers = params['layers']
    scale = 0.25
    lp0 = layers[0]
    q, k, v = _qkv_call(
        h, lp0['Wq'] * scale, lp0['Wk'], lp0['Wv'],
        (lp0['bq'] * scale).reshape(1, HID), lp0['bk'].reshape(1, HID),
        lp0['bv'].reshape(1, HID))

    for li, lp in enumerate(layers):
        accs = _edge_attn(q, k, v, src_p, dst_p)
        n0 = accs[0, :, :HID]
        d0 = accs[0, :, HID:HID + 16]
        n1 = accs[1, :, :HID]
        d1 = accs[1, :, HID:HID + 16]
        lp_perm = dict(lp)
        lp_perm['Wo'] = lp['Wo'][o_idx, :]
        if li < len(layers) - 1:
            h, q, k, v = _post_qkv_call(n0, d0, n1, d1, h, rexp, lp_perm,
                                        layers[li + 1])
        else:
            out = _post_readout_call(n0, d0, n1, d1, h, rexp, lp_perm,
                                     params['mlp'])
    return out[:N_NODES, :NCLS]


# speedup vs baseline: 1.0088x; 1.0088x over previous
import functools

import numpy as _np

import jax
import jax.numpy as jnp
from jax import lax
from jax.experimental import pallas as pl
from jax.experimental.pallas import tpu as pltpu
from jax.experimental.pallas import tpu_sc as plsc

N_NODES = 10000
NPAD = 10240
E_EDGES = 320000
HID = 128
HEADS = 8
DH = 16
WACC = 144
NCLS = 10

NC = 2
NS = 16
EPT = 10112
EPAD = EPT * NC * NS
CHK = 64
IDXB = 2
NCHUNK = EPT // CHK
ROWS_PT = NPAD // NS

_f32 = jnp.float32
_i32 = jnp.int32

_sc_mesh = plsc.VectorSubcoreMesh(core_axis_name="c", subcore_axis_name="s")


def _shuf(v, idx):
    dnums = lax.GatherDimensionNumbers(
        offset_dims=(), collapsed_slice_dims=(0,), start_index_map=(0,))
    return lax.gather(v, idx[:, None], dnums, (1,),
                      mode=lax.GatherScatterMode.PROMISE_IN_BOUNDS)


@functools.partial(
    pl.kernel,
    out_type=jax.ShapeDtypeStruct((NPAD, HID), _f32),
    mesh=_sc_mesh,
    scratch_types=[
        pltpu.VMEM((80,), _i32),
        pltpu.VMEM((80, HID), _f32),
        pltpu.SemaphoreType.DMA,
    ],
)
def _emb_gather(nf_hbm, emb_hbm, out_hbm, idx_v, rows_v, sem):
    wid = lax.axis_index("c") * NS + lax.axis_index("s")
    for j in range(4):
        base = wid * 320 + j * 80
        pltpu.sync_copy(nf_hbm.at[pl.ds(base, 80)], idx_v)
        pltpu.async_copy(emb_hbm.at[idx_v], rows_v, sem).wait()
        pltpu.sync_copy(rows_v, out_hbm.at[pl.ds(base, 80)])


@functools.partial(
    pl.kernel,
    out_type=jax.ShapeDtypeStruct((NC, NPAD, WACC), _f32),
    mesh=_sc_mesh,
    scratch_types=[
        pltpu.VMEM((2, IDXB, CHK), _i32),
        pltpu.VMEM((2, IDXB, CHK), _i32),
        pltpu.VMEM((2, CHK, HID), jnp.bfloat16),
        pltpu.VMEM((2, CHK, HID), jnp.bfloat16),
        pltpu.VMEM((2, CHK, HID), jnp.bfloat16),
        pltpu.VMEM((CHK, WACC), _f32),
        pltpu.VMEM_SHARED((NPAD, WACC), _f32),
        pltpu.SemaphoreType.DMA,
        pltpu.SemaphoreType.DMA,
    ],
    compiler_params=pltpu.CompilerParams(use_tc_tiling_on_sc=False, needs_layout_passes=False),
)
def _edge_attn(q_hbm, k_hbm, v_hbm, src_hbm, dst_hbm, out_hbm,
               sidx, didx, kbuf, qbuf, vbuf, wbuf, acc, gsem0, gsem1):
    cid = lax.axis_index("c")
    sid = lax.axis_index("s")
    zero16 = jnp.zeros((16,), _f32)
    iota16 = lax.iota(_i32, 16)
    perms = [jnp.bitwise_xor(iota16, k2) for k2 in (4, 2, 1)]
    ppick = (iota16 & 1) * 8
    pmask = [lax.shift_right_logical(iota16, 1) == g for g in range(4)]
    gsem = (gsem0, gsem1)

    row0 = sid * ROWS_PT
    rbase = (cid * NS + sid) * NCHUNK

    def _stage(sl, row, b):
        pltpu.async_copy(k_hbm.at[sidx.at[sl, row]], kbuf.at[b], gsem[b])
        pltpu.async_copy(q_hbm.at[didx.at[sl, row]], qbuf.at[b], gsem[b])
        pltpu.async_copy(v_hbm.at[sidx.at[sl, row]], vbuf.at[b], gsem[b])

    def _drain(b):
        pltpu.make_async_copy(k_hbm.at[pl.ds(0, CHK)], kbuf.at[b], gsem[b]).wait()
        pltpu.make_async_copy(q_hbm.at[pl.ds(0, CHK)], qbuf.at[b], gsem[b]).wait()
        pltpu.make_async_copy(v_hbm.at[pl.ds(0, CHK)], vbuf.at[b], gsem[b]).wait()

    def _load_idx(jb, sl):
        pltpu.sync_copy(src_hbm.at[pl.ds(rbase + jb * IDXB, IDXB)], sidx.at[sl])
        pltpu.sync_copy(dst_hbm.at[pl.ds(rbase + jb * IDXB, IDXB)], didx.at[sl])
    _load_idx(0, 0)
    _stage(0, 0, 0)

    def _zrow(r, carry):
        for j in range(WACC // 16):
            wbuf[r, pl.ds(j * 16, 16)] = zero16
        return carry
    lax.fori_loop(0, CHK, _zrow, 0)
    def _zacc(j, carry):
        pltpu.sync_copy(wbuf, acc.at[pl.ds(row0 + j * CHK, CHK)])
        return carry
    lax.fori_loop(0, ROWS_PT // CHK, _zacc, 0)
    plsc.subcore_barrier()


    def _batch(jb, carry):
        sl = jb % 2
        nsl = 1 - sl
        _load_idx(jb + 1, nsl)
        for j in range(IDXB):
            b = j & 1
            if j < IDXB - 1:
                _stage(sl, j + 1, 1 - b)
            else:
                _stage(nsl, 0, 1 - b)
            _drain(b)

            @plsc.parallel_loop(0, CHK, 1, unroll=4)
            def _edge(e):
                drow = jnp.zeros((16,), _f32)
                for g in range(HEADS // 2):
                    c0 = g * 32
                    klo, khi = plsc.unpack(kbuf[b, e, pl.ds(c0, 32)],
                                           format=plsc.PackFormat.INTERLEAVED,
                                           preferred_element_type=_f32)
                    qlo, qhi = plsc.unpack(qbuf[b, e, pl.ds(c0, 32)],
                                           format=plsc.PackFormat.INTERLEAVED,
                                           preferred_element_type=_f32)
                    sv = klo * qlo + khi * qhi
                    for p2 in perms:
                        sv = sv + _shuf(sv, p2)
                    sv = jnp.exp(jnp.clip(sv, -5.0, 5.0))
                    vlo, vhi = plsc.unpack(vbuf[b, e, pl.ds(c0, 32)],
                                           format=plsc.PackFormat.INTERLEAVED,
                                           preferred_element_type=_f32)
                    wbuf[e, pl.ds(c0, 16)] = sv * vlo
                    wbuf[e, pl.ds(c0 + 16, 16)] = sv * vhi
                    drow = jnp.where(pmask[g], _shuf(sv, ppick), drow)
                wbuf[e, pl.ds(HID, 16)] = drow
            pltpu.sync_copy(wbuf, acc.at[didx.at[sl, j]], add=True)
        return carry
    lax.fori_loop(0, NCHUNK // IDXB, _batch, 0)
    _drain(0)

    plsc.subcore_barrier()
    def _wout(j, carry):
        r = row0 + j * CHK
        pltpu.sync_copy(acc.at[pl.ds(r, CHK)], out_hbm.at[cid, pl.ds(r, CHK)])
        return carry
    lax.fori_loop(0, ROWS_PT // CHK, _wout, 0)


BLK = 1024


def _qkv_body(h_ref, wq, wk, wv, bq, bk, bv, q_ref, k_ref, v_ref):
    h = h_ref[...]
    bf = jnp.bfloat16
    q_ref[...] = (jnp.dot(h, wq[...], preferred_element_type=_f32) + bq[...]).astype(bf)
    k_ref[...] = (jnp.dot(h, wk[...], preferred_element_type=_f32) + bk[...]).astype(bf)
    v_ref[...] = (jnp.dot(h, wv[...], preferred_element_type=_f32) + bv[...]).astype(bf)


def _qkv_call(h, wq, wk, wv, bq, bk, bv):
    grid = (NPAD // BLK,)
    row_spec = pl.BlockSpec((BLK, HID), lambda i: (i, 0))
    w_spec = pl.BlockSpec((HID, HID), lambda i: (0, 0))
    b_spec = pl.BlockSpec((1, HID), lambda i: (0, 0))
    return pl.pallas_call(
        _qkv_body,
        grid=grid,
        in_specs=[row_spec, w_spec, w_spec, w_spec, b_spec, b_spec, b_spec],
        out_specs=[row_spec, row_spec, row_spec],
        out_shape=[jax.ShapeDtypeStruct((NPAD, HID), jnp.bfloat16)] * 3,
    )(h, wq, wk, wv, bq, bk, bv)


def _ln(x, g, b):
    mu = jnp.mean(x, axis=1, keepdims=True)
    xc = x - mu
    var = jnp.mean(xc * xc, axis=1, keepdims=True)
    return xc * lax.rsqrt(var + 1e-5) * g + b


def _post_h(n0, d0, n1, d1, hin, rexp, wo, bo, w1, b1, w2, b2, g1, be1, g2, be2):
    numer = n0[...] + n1[...]
    den = d0[...] + d1[...] + 1e-6
    denexp = jnp.dot(den, rexp[...], preferred_element_type=_f32)
    attn = numer / denexp
    h = hin[...] + jnp.dot(attn, wo[...], preferred_element_type=_f32) + bo[...]
    h = _ln(h, g1[...], be1[...])
    ff = jnp.maximum(jnp.dot(h, w1[...], preferred_element_type=_f32) + b1[...], 0.0)
    h2 = h + jnp.dot(ff, w2[...], preferred_element_type=_f32) + b2[...]
    return _ln(h2, g2[...], be2[...])


def _post_qkv_body(n0, d0, n1, d1, hin, rexp, wo, bo, w1, b1, w2, b2, g1, be1,
                   g2, be2, wq, wk, wv, bq, bk, bv, h_ref, q_ref, k_ref, v_ref):
    h = _post_h(n0, d0, n1, d1, hin, rexp, wo, bo, w1, b1, w2, b2, g1, be1, g2, be2)
    h_ref[...] = h
    bf = jnp.bfloat16
    q_ref[...] = (jnp.dot(h, wq[...], preferred_element_type=_f32) + bq[...]).astype(bf)
    k_ref[...] = (jnp.dot(h, wk[...], preferred_element_type=_f32) + bk[...]).astype(bf)
    v_ref[...] = (jnp.dot(h, wv[...], preferred_element_type=_f32) + bv[...]).astype(bf)


def _post_readout_body(n0, d0, n1, d1, hin, rexp, wo, bo, w1, b1, w2, b2, g1, be1,
                       g2, be2, m0, c0, m1, c1, m2, c2, out_ref):
    h = _post_h(n0, d0, n1, d1, hin, rexp, wo, bo, w1, b1, w2, b2, g1, be1, g2, be2)
    x = jnp.maximum(jnp.dot(h, m0[...], preferred_element_type=_f32) + c0[...], 0.0)
    x = jnp.maximum(jnp.dot(x, m1[...], preferred_element_type=_f32) + c1[...], 0.0)
    out_ref[...] = jnp.dot(x, m2[...], preferred_element_type=_f32) + c2[...]


def _post_args(lp):
    return (lp['Wo'], lp['bo'].reshape(1, HID), lp['W1'], lp['b1'].reshape(1, 2 * HID),
            lp['W2'], lp['b2'].reshape(1, HID), lp['g1'].reshape(1, HID),
            lp['be1'].reshape(1, HID), lp['g2'].reshape(1, HID), lp['be2'].reshape(1, HID))


def _post_specs():
    row_spec = pl.BlockSpec((BLK, HID), lambda i: (i, 0))
    den_spec = pl.BlockSpec((BLK, 16), lambda i: (i, 0))
    rexp_spec = pl.BlockSpec((16, HID), lambda i: (0, 0))
    w_spec = pl.BlockSpec((HID, HID), lambda i: (0, 0))
    w1_spec = pl.BlockSpec((HID, 2 * HID), lambda i: (0, 0))
    w2_spec = pl.BlockSpec((2 * HID, HID), lambda i: (0, 0))
    b_spec = pl.BlockSpec((1, HID), lambda i: (0, 0))
    b1_spec = pl.BlockSpec((1, 2 * HID), lambda i: (0, 0))
    return [row_spec, den_spec, row_spec, den_spec, row_spec, rexp_spec,
            w_spec, b_spec, w1_spec, b1_spec, w2_spec, b_spec,
            b_spec, b_spec, b_spec, b_spec]


def _post_qkv_call(n0, d0, n1, d1, hin, rexp, lp, nlp):
    grid = (NPAD // BLK,)
    row_spec = pl.BlockSpec((BLK, HID), lambda i: (i, 0))
    w_spec = pl.BlockSpec((HID, HID), lambda i: (0, 0))
    b_spec = pl.BlockSpec((1, HID), lambda i: (0, 0))
    scale = 0.25
    return pl.pallas_call(
        _post_qkv_body,
        grid=grid,
        in_specs=_post_specs() + [w_spec, w_spec, w_spec, b_spec, b_spec, b_spec],
        out_specs=[row_spec, row_spec, row_spec, row_spec],
        out_shape=[jax.ShapeDtypeStruct((NPAD, HID), _f32)] +
                  [jax.ShapeDtypeStruct((NPAD, HID), jnp.bfloat16)] * 3,
    )(n0, d0, n1, d1, hin, rexp, *_post_args(lp),
      nlp['Wq'] * scale, nlp['Wk'], nlp['Wv'],
      (nlp['bq'] * scale).reshape(1, HID), nlp['bk'].reshape(1, HID),
      nlp['bv'].reshape(1, HID))


def _post_readout_call(n0, d0, n1, d1, hin, rexp, lp, mlp):
    grid = (NPAD // BLK,)
    row_spec = pl.BlockSpec((BLK, HID), lambda i: (i, 0))
    w_spec = pl.BlockSpec((HID, HID), lambda i: (0, 0))
    b_spec = pl.BlockSpec((1, HID), lambda i: (0, 0))
    (w0, b0), (w1, b1), (w2, b2) = mlp
    return pl.pallas_call(
        _post_readout_body,
        grid=grid,
        in_specs=_post_specs() + [w_spec, b_spec, w_spec, b_spec, w_spec, b_spec],
        out_specs=row_spec,
        out_shape=jax.ShapeDtypeStruct((NPAD, HID), _f32),
    )(n0, d0, n1, d1, hin, rexp, *_post_args(lp),
      _pad_mat(w0, HID, HID), _pad_mat(b0.reshape(1, -1), 1, HID),
      _pad_mat(w1, HID, HID), _pad_mat(b1.reshape(1, -1), 1, HID),
      _pad_mat(w2, HID, HID), _pad_mat(b2.reshape(1, -1), 1, HID))


def _pad_mat(w, rows, cols):
    r, c = w.shape
    return jnp.pad(w, ((0, rows - r), (0, cols - c)))


def kernel(node_feat, edge_index, params):
    src = edge_index[0]
    dst = edge_index[1]
    pad_n = EPAD + IDXB * CHK - E_EDGES
    src_p = jnp.concatenate([src, jnp.full((pad_n,), NPAD - 1, _i32)]).reshape(-1, CHK)
    dst_p = jnp.concatenate([dst, jnp.full((pad_n,), NPAD - 1, _i32)]).reshape(-1, CHK)
    nf_p = jnp.concatenate([node_feat, jnp.zeros((NPAD - N_NODES,), _i32)])

    c = _np.arange(HID)
    g2 = c // 32
    r = c % 32
    head = 2 * g2 + ((r // 8) & 1)
    feat = 2 * (r % 8) + (r // 16)
    o_idx = head * DH + feat
    rexp = jnp.asarray((_np.arange(16)[:, None] == head[None, :]).astype(_np.float32))

    h = _emb_gather(nf_p, params['emb'])

    layers = params['layers']
    scale = 0.25
    lp0 = layers[0]
    q, k, v = _qkv_call(
        h, lp0['Wq'] * scale, lp0['Wk'], lp0['Wv'],
        (lp0['bq'] * scale).reshape(1, HID), lp0['bk'].reshape(1, HID),
        lp0['bv'].reshape(1, HID))

    for li, lp in enumerate(layers):
        accs = _edge_attn(q, k, v, src_p, dst_p)
        n0 = accs[0, :, :HID]
        d0 = accs[0, :, HID:HID + 16]
        n1 = accs[1, :, :HID]
        d1 = accs[1, :, HID:HID + 16]
        lp_perm = dict(lp)
        lp_perm['Wo'] = lp['Wo'][o_idx, :]
        if li < len(layers) - 1:
            h, q, k, v = _post_qkv_call(n0, d0, n1, d1, h, rexp, lp_perm,
                                        layers[li + 1])
        else:
            out = _post_readout_call(n0, d0, n1, d1, h, rexp, lp_perm,
                                     params['mlp'])
    return out[:N_NODES, :NCLS]
